# baseline (speedup 1.0000x reference)
import functools

import jax
import jax.numpy as jnp
from jax import lax
from jax.experimental import pallas as pl
from jax.experimental.pallas import tpu as pltpu
from jax.experimental.pallas import tpu_sc as plsc

N = 10000
E = 320000
D = 128

NC = 2
NS = 16
NW = NC * NS
EPW = E // NW

KD = 80
NCHD = EPW // KD

KC = 80
EPWP = EPW
NCH = EPWP // KC
NGRP = NCH // 2

RPT = 624
RLAST = N - RPT * (NS - 1)

_mesh = plsc.VectorSubcoreMesh(core_axis_name="c", subcore_axis_name="s")


@functools.partial(
    pl.kernel,
    out_type=jax.ShapeDtypeStruct((NC * N,), jnp.float32),
    mesh=_mesh,
    scratch_types=[
        pltpu.VMEM((NCHD, KD), jnp.int32),
        pltpu.VMEM((KD,), jnp.float32),
        pltpu.VMEM((RLAST,), jnp.float32),
        pltpu.VMEM_SHARED((N,), jnp.float32),
    ],
)
def _deg_kernel(dst3d, out, idx_v, ones_v, stage_v, deg_sh):
    cid = lax.axis_index("c")
    sid = lax.axis_index("s")
    wid = cid * NS + sid
    for i in range(KD // 16):
        ones_v[pl.ds(i * 16, 16)] = jnp.ones((16,), jnp.float32)
    for i in range(RLAST // 16):
        stage_v[pl.ds(i * 16, 16)] = jnp.zeros((16,), jnp.float32)
    pltpu.sync_copy(stage_v.at[pl.ds(0, RPT)], deg_sh.at[pl.ds(sid * RPT, RPT)])

    @pl.when(sid == NS - 1)
    def _():
        pltpu.sync_copy(stage_v.at[pl.ds(RPT, RLAST - RPT)],
                        deg_sh.at[pl.ds(RPT * NS, RLAST - RPT)])

    pltpu.sync_copy(dst3d.at[wid], idx_v)
    plsc.subcore_barrier()

    def step(j, carry):
        pltpu.sync_copy(ones_v, deg_sh.at[idx_v.at[j]], add=True)
        return carry

    lax.fori_loop(0, NCHD, step, 0)
    plsc.subcore_barrier()
    pltpu.sync_copy(deg_sh.at[pl.ds(sid * RPT, RPT)], stage_v.at[pl.ds(0, RPT)])
    pltpu.sync_copy(stage_v.at[pl.ds(0, RPT)],
                    out.at[pl.ds(cid * N + sid * RPT, RPT)])

    @pl.when(sid == NS - 1)
    def _():
        pltpu.sync_copy(deg_sh.at[pl.ds(RPT * NS, RLAST - RPT)],
                        stage_v.at[pl.ds(RPT, RLAST - RPT)])
        pltpu.sync_copy(stage_v.at[pl.ds(RPT, RLAST - RPT)],
                        out.at[pl.ds(cid * N + RPT * NS, RLAST - RPT)])


@functools.partial(
    pl.kernel,
    out_type=jax.ShapeDtypeStruct((NC, N, D), jnp.float32),
    mesh=_mesh,
    scratch_types=[
        pltpu.VMEM((EPWP,), jnp.int32),
        pltpu.VMEM((NCH, KC), jnp.int32),
        pltpu.VMEM((2, KC, D), jnp.float32),
        pltpu.VMEM_SHARED((N, D), jnp.float32),
        pltpu.SemaphoreType.DMA,
        pltpu.SemaphoreType.DMA,
        pltpu.SemaphoreType.DMA,
    ],
)
def _gather_scatter_kernel(z, srcp, dstp3, zerosd, out,
                           src_v, didx_v, rows_v, acc_sh,
                           gsem, ssem0, ssem1):
    cid = lax.axis_index("c")
    sid = lax.axis_index("s")
    wid = cid * NS + sid
    pltpu.sync_copy(zerosd.at[pl.ds(0, RPT)], acc_sh.at[pl.ds(sid * RPT, RPT)])

    @pl.when(sid == NS - 1)
    def _():
        pltpu.sync_copy(zerosd.at[pl.ds(RPT, RLAST - RPT)],
                        acc_sh.at[pl.ds(RPT * NS, RLAST - RPT)])

    pltpu.sync_copy(srcp.at[wid], src_v)
    pltpu.sync_copy(dstp3.at[wid], didx_v)
    plsc.subcore_barrier()

    ssems = (ssem0, ssem1)

    def body(gg, carry):
        gds = []
        for h in range(2):
            g = gg * 2 + h
            @pl.when(g >= 2)
            def _():
                pltpu.make_async_copy(z.at[pl.ds(0, KC)], rows_v.at[h],
                                      ssems[h]).wait()
            off = pl.multiple_of(g * KC, 8)
            gds.append(pltpu.async_copy(
                z.at[src_v.at[pl.ds(off, KC)]], rows_v.at[h], gsem))
        for h in range(2):
            g = gg * 2 + h
            gds[h].wait()
            pltpu.async_copy(rows_v.at[h], acc_sh.at[didx_v.at[g]],
                             ssems[h], add=True)
        return carry

    lax.fori_loop(0, NGRP, body, 0)
    pltpu.make_async_copy(z.at[pl.ds(0, KC)], rows_v.at[0], ssem0).wait()
    j = NCH - 1
    pltpu.async_copy(z.at[src_v.at[pl.ds(j * KC, KC)]], rows_v.at[0],
                     gsem).wait()
    pltpu.sync_copy(rows_v.at[0], acc_sh.at[didx_v.at[j]], add=True)
    pltpu.make_async_copy(z.at[pl.ds(0, KC)], rows_v.at[1], ssem1).wait()
    plsc.subcore_barrier()
    pltpu.sync_copy(acc_sh.at[pl.ds(sid * RPT, RPT)],
                    out.at[cid, pl.ds(sid * RPT, RPT)])

    @pl.when(sid == NS - 1)
    def _():
        pltpu.sync_copy(acc_sh.at[pl.ds(RPT * NS, RLAST - RPT)],
                        out.at[cid, pl.ds(RPT * NS, RLAST - RPT)])


def _mm_body(x_ref, wt_ref, b_ref, y_ref):
    y = jnp.dot(x_ref[...], wt_ref[...], preferred_element_type=jnp.float32)
    y_ref[...] = y + b_ref[...]


def _scale_body(y_ref, deg_ref, z_ref):
    deg = deg_ref[...]
    d = deg[0] + deg[1] + 1.0
    z_ref[...] = y_ref[...] * lax.rsqrt(d)


def _final_body(acc_ref, z_ref, deg_ref, o_ref):
    deg = deg_ref[...]
    d = deg[0] + deg[1] + 1.0
    norm = lax.rsqrt(d)
    acc = acc_ref[...]
    s = (acc[0] + acc[1] + z_ref[...]) * norm
    o_ref[...] = jnp.maximum(s, 0.0)


def kernel(X, edge_index, W, b):
    src = edge_index[0].astype(jnp.int32)
    dst = edge_index[1].astype(jnp.int32)
    dst3d = dst.reshape(NW, NCHD, KD)

    srcp = src.reshape(NW, EPW)
    dstp3 = dst.reshape(NW, NCH, KC)

    zerosd = jnp.zeros((RLAST, D), jnp.float32)

    degflat = _deg_kernel(dst3d)
    deg = degflat.reshape(NC, N, 1)

    y = pl.pallas_call(
        _mm_body,
        out_shape=jax.ShapeDtypeStruct((N, D), jnp.float32),
    )(X, W.T, b.reshape(1, D))

    z = pl.pallas_call(
        _scale_body,
        out_shape=jax.ShapeDtypeStruct((N, D), jnp.float32),
    )(y, deg)

    acc = _gather_scatter_kernel(z, srcp, dstp3, zerosd)

    out = pl.pallas_call(
        _final_body,
        out_shape=jax.ShapeDtypeStruct((N, D), jnp.float32),
    )(acc, z, deg)
    return out


# speedup vs baseline: 1.0734x; 1.0734x over previous
import functools

import jax
import jax.numpy as jnp
from jax import lax
from jax.experimental import pallas as pl
from jax.experimental.pallas import tpu as pltpu
from jax.experimental.pallas import tpu_sc as plsc

N = 10000
E = 320000
D = 128

NC = 2
NS = 16
NW = NC * NS
EPW = E // NW

KD = 80
NCHD = EPW // KD

KC = 80
EPWP = EPW
NCH = EPWP // KC
NGRP = NCH // 2

RPT = 624
RLAST = N - RPT * (NS - 1)

_mesh = plsc.VectorSubcoreMesh(core_axis_name="c", subcore_axis_name="s")


@functools.partial(
    pl.kernel,
    out_type=jax.ShapeDtypeStruct((NC * N,), jnp.float32),
    mesh=_mesh,
    scratch_types=[
        pltpu.VMEM((NCHD, KD), jnp.int32),
        pltpu.VMEM((KD,), jnp.float32),
        pltpu.VMEM((RLAST,), jnp.float32),
        pltpu.VMEM_SHARED((N,), jnp.float32),
    ],
)
def _deg_kernel(dst3d, out, idx_v, ones_v, stage_v, deg_sh):
    cid = lax.axis_index("c")
    sid = lax.axis_index("s")
    wid = cid * NS + sid
    for i in range(KD // 16):
        ones_v[pl.ds(i * 16, 16)] = jnp.ones((16,), jnp.float32)
    for i in range(RLAST // 16):
        stage_v[pl.ds(i * 16, 16)] = jnp.zeros((16,), jnp.float32)
    pltpu.sync_copy(stage_v.at[pl.ds(0, RPT)], deg_sh.at[pl.ds(sid * RPT, RPT)])

    @pl.when(sid == NS - 1)
    def _():
        pltpu.sync_copy(stage_v.at[pl.ds(RPT, RLAST - RPT)],
                        deg_sh.at[pl.ds(RPT * NS, RLAST - RPT)])

    pltpu.sync_copy(dst3d.at[wid], idx_v)
    plsc.subcore_barrier()

    def step(j, carry):
        pltpu.sync_copy(ones_v, deg_sh.at[idx_v.at[j]], add=True)
        return carry

    lax.fori_loop(0, NCHD, step, 0)
    plsc.subcore_barrier()
    pltpu.sync_copy(deg_sh.at[pl.ds(sid * RPT, RPT)], stage_v.at[pl.ds(0, RPT)])
    pltpu.sync_copy(stage_v.at[pl.ds(0, RPT)],
                    out.at[pl.ds(cid * N + sid * RPT, RPT)])

    @pl.when(sid == NS - 1)
    def _():
        pltpu.sync_copy(deg_sh.at[pl.ds(RPT * NS, RLAST - RPT)],
                        stage_v.at[pl.ds(RPT, RLAST - RPT)])
        pltpu.sync_copy(stage_v.at[pl.ds(RPT, RLAST - RPT)],
                        out.at[pl.ds(cid * N + RPT * NS, RLAST - RPT)])


@functools.partial(
    pl.kernel,
    out_type=jax.ShapeDtypeStruct((NC, N, D), jnp.float32),
    mesh=_mesh,
    scratch_types=[
        pltpu.VMEM((EPWP,), jnp.int32),
        pltpu.VMEM((EPWP,), jnp.int32),
        pltpu.VMEM((3, KC, D), jnp.float32),
        pltpu.VMEM_SHARED((N, D), jnp.float32),
        pltpu.SemaphoreType.DMA,
        pltpu.SemaphoreType.DMA,
        pltpu.SemaphoreType.DMA,
        pltpu.SemaphoreType.DMA,
    ],
)
def _gather_scatter_kernel(z, srcp, dstp2, zerosd, out,
                           src_v, didx_v, rows_v, acc_sh,
                           gsem, ssem0, ssem1, ssem2):
    cid = lax.axis_index("c")
    sid = lax.axis_index("s")
    wid = cid * NS + sid
    pltpu.sync_copy(zerosd.at[pl.ds(0, RPT)], acc_sh.at[pl.ds(sid * RPT, RPT)])

    @pl.when(sid == NS - 1)
    def _():
        pltpu.sync_copy(zerosd.at[pl.ds(RPT, RLAST - RPT)],
                        acc_sh.at[pl.ds(RPT * NS, RLAST - RPT)])

    pltpu.sync_copy(srcp.at[wid], src_v)
    pltpu.sync_copy(dstp2.at[wid], didx_v)
    plsc.subcore_barrier()

    ssems = (ssem0, ssem1, ssem2)
    NSLOT = 3

    def body(gg, carry):
        gds = []
        for h in range(NSLOT):
            g = gg * NSLOT + h
            @pl.when(g >= NSLOT)
            def _():
                pltpu.make_async_copy(z.at[pl.ds(0, KC)], rows_v.at[h],
                                      ssems[h]).wait()
            off = pl.multiple_of(g * KC, 8)
            gds.append(pltpu.async_copy(
                z.at[src_v.at[pl.ds(off, KC)]], rows_v.at[h], gsem))
        for h in range(NSLOT):
            g = gg * NSLOT + h
            goff = pl.multiple_of(g * KC, 8)
            gds[h].wait()
            pltpu.async_copy(rows_v.at[h],
                             acc_sh.at[didx_v.at[pl.ds(goff, KC)]],
                             ssems[h], add=True)
        return carry

    nbody = NCH // NSLOT
    lax.fori_loop(0, nbody, body, 0)
    for t in range(NCH - nbody * NSLOT):
        j = nbody * NSLOT + t
        pltpu.make_async_copy(z.at[pl.ds(0, KC)], rows_v.at[t], ssems[t]).wait()
        pltpu.async_copy(z.at[src_v.at[pl.ds(j * KC, KC)]], rows_v.at[t],
                         gsem).wait()
        pltpu.sync_copy(rows_v.at[t], acc_sh.at[didx_v.at[pl.ds(j * KC, KC)]],
                        add=True)
    pltpu.make_async_copy(z.at[pl.ds(0, KC)], rows_v.at[2], ssem2).wait()
    plsc.subcore_barrier()
    pltpu.sync_copy(acc_sh.at[pl.ds(sid * RPT, RPT)],
                    out.at[cid, pl.ds(sid * RPT, RPT)])

    @pl.when(sid == NS - 1)
    def _():
        pltpu.sync_copy(acc_sh.at[pl.ds(RPT * NS, RLAST - RPT)],
                        out.at[cid, pl.ds(RPT * NS, RLAST - RPT)])


def _mm_body(x_ref, wt_ref, b_ref, y_ref):
    y = jnp.dot(x_ref[...], wt_ref[...], preferred_element_type=jnp.float32)
    y_ref[...] = y + b_ref[...]


def _scale_body(y_ref, deg_ref, z_ref):
    deg = deg_ref[...]
    d = deg[0] + deg[1] + 1.0
    z_ref[...] = y_ref[...] * lax.rsqrt(d)


def _final_body(acc_ref, z_ref, deg_ref, o_ref):
    deg = deg_ref[...]
    d = deg[0] + deg[1] + 1.0
    norm = lax.rsqrt(d)
    acc = acc_ref[...]
    s = (acc[0] + acc[1] + z_ref[...]) * norm
    o_ref[...] = jnp.maximum(s, 0.0)


def kernel(X, edge_index, W, b):
    src = edge_index[0].astype(jnp.int32)
    dst = edge_index[1].astype(jnp.int32)
    dst3d = dst.reshape(NW, NCHD, KD)

    srcp = src.reshape(NW, EPW)
    dstp2 = dst.reshape(NW, EPW)

    zerosd = jnp.zeros((RLAST, D), jnp.float32)

    degflat = _deg_kernel(dst3d)
    deg = degflat.reshape(NC, N, 1)

    y = pl.pallas_call(
        _mm_body,
        out_shape=jax.ShapeDtypeStruct((N, D), jnp.float32),
    )(X, W.T, b.reshape(1, D))

    z = pl.pallas_call(
        _scale_body,
        out_shape=jax.ShapeDtypeStruct((N, D), jnp.float32),
    )(y, deg)

    acc = _gather_scatter_kernel(z, srcp, dstp2, zerosd)

    out = pl.pallas_call(
        _final_body,
        out_shape=jax.ShapeDtypeStruct((N, D), jnp.float32),
    )(acc, z, deg)
    return out


# speedup vs baseline: 1.0750x; 1.0015x over previous
import functools

import jax
import jax.numpy as jnp
from jax import lax
from jax.experimental import pallas as pl
from jax.experimental.pallas import tpu as pltpu
from jax.experimental.pallas import tpu_sc as plsc

N = 10000
E = 320000
D = 128

NC = 2
NS = 16
NW = NC * NS
EPW = E // NW

KD = 80
NCHD = EPW // KD

KC = 80
EPWP = EPW
NCH = EPWP // KC
NGRP = NCH // 2

RPT = 624
RLAST = N - RPT * (NS - 1)

_mesh = plsc.VectorSubcoreMesh(core_axis_name="c", subcore_axis_name="s")


@functools.partial(
    pl.kernel,
    out_type=jax.ShapeDtypeStruct((NC * N,), jnp.float32),
    mesh=_mesh,
    scratch_types=[
        pltpu.VMEM((NCHD, KD), jnp.int32),
        pltpu.VMEM((KD,), jnp.float32),
        pltpu.VMEM((RLAST,), jnp.float32),
        pltpu.VMEM_SHARED((N,), jnp.float32),
    ],
)
def _deg_kernel(dst3d, out, idx_v, ones_v, stage_v, deg_sh):
    cid = lax.axis_index("c")
    sid = lax.axis_index("s")
    wid = cid * NS + sid
    for i in range(KD // 16):
        ones_v[pl.ds(i * 16, 16)] = jnp.ones((16,), jnp.float32)
    for i in range(RLAST // 16):
        stage_v[pl.ds(i * 16, 16)] = jnp.zeros((16,), jnp.float32)
    pltpu.sync_copy(stage_v.at[pl.ds(0, RPT)], deg_sh.at[pl.ds(sid * RPT, RPT)])

    @pl.when(sid == NS - 1)
    def _():
        pltpu.sync_copy(stage_v.at[pl.ds(RPT, RLAST - RPT)],
                        deg_sh.at[pl.ds(RPT * NS, RLAST - RPT)])

    pltpu.sync_copy(dst3d.at[wid], idx_v)
    plsc.subcore_barrier()

    def step(j, carry):
        pltpu.sync_copy(ones_v, deg_sh.at[idx_v.at[j]], add=True)
        return carry

    lax.fori_loop(0, NCHD, step, 0)
    plsc.subcore_barrier()
    pltpu.sync_copy(deg_sh.at[pl.ds(sid * RPT, RPT)], stage_v.at[pl.ds(0, RPT)])
    pltpu.sync_copy(stage_v.at[pl.ds(0, RPT)],
                    out.at[pl.ds(cid * N + sid * RPT, RPT)])

    @pl.when(sid == NS - 1)
    def _():
        pltpu.sync_copy(deg_sh.at[pl.ds(RPT * NS, RLAST - RPT)],
                        stage_v.at[pl.ds(RPT, RLAST - RPT)])
        pltpu.sync_copy(stage_v.at[pl.ds(RPT, RLAST - RPT)],
                        out.at[pl.ds(cid * N + RPT * NS, RLAST - RPT)])


@functools.partial(
    pl.kernel,
    out_type=jax.ShapeDtypeStruct((NC, N, D), jnp.float32),
    mesh=_mesh,
    scratch_types=[
        pltpu.VMEM((EPWP,), jnp.int32),
        pltpu.VMEM((EPWP,), jnp.int32),
        pltpu.VMEM((3, KC, D), jnp.float32),
        pltpu.VMEM_SHARED((N, D), jnp.float32),
        pltpu.SemaphoreType.DMA,
        pltpu.SemaphoreType.DMA,
        pltpu.SemaphoreType.DMA,
        pltpu.SemaphoreType.DMA,
    ],
)
def _gather_scatter_kernel(z, srcp, dstp2, zerosd, out,
                           src_v, didx_v, rows_v, acc_sh,
                           gsem, ssem0, ssem1, ssem2):
    cid = lax.axis_index("c")
    sid = lax.axis_index("s")
    wid = cid * NS + sid
    pltpu.sync_copy(zerosd.at[pl.ds(0, RPT)], acc_sh.at[pl.ds(sid * RPT, RPT)])

    @pl.when(sid == NS - 1)
    def _():
        pltpu.sync_copy(zerosd.at[pl.ds(RPT, RLAST - RPT)],
                        acc_sh.at[pl.ds(RPT * NS, RLAST - RPT)])

    pltpu.sync_copy(srcp.at[wid], src_v)
    pltpu.sync_copy(dstp2.at[wid], didx_v)
    plsc.subcore_barrier()

    ssems = (ssem0, ssem1, ssem2)
    NSLOT = 3

    def body(gg, carry):
        gds = []
        for h in range(NSLOT):
            g = gg * NSLOT + h
            @pl.when(g >= NSLOT)
            def _():
                pltpu.make_async_copy(z.at[pl.ds(0, KC)], rows_v.at[h],
                                      ssems[h]).wait()
            off = pl.multiple_of(g * KC, 8)
            gds.append(pltpu.async_copy(
                z.at[src_v.at[pl.ds(off, KC)]], rows_v.at[h], gsem))
        for h in range(NSLOT):
            g = gg * NSLOT + h
            goff = pl.multiple_of(g * KC, 8)
            gds[h].wait()
            pltpu.async_copy(rows_v.at[h],
                             acc_sh.at[didx_v.at[pl.ds(goff, KC)]],
                             ssems[h], add=True)
        return carry

    nbody = NCH // NSLOT
    lax.fori_loop(0, nbody, body, 0)
    for t in range(NCH - nbody * NSLOT):
        j = nbody * NSLOT + t
        pltpu.make_async_copy(z.at[pl.ds(0, KC)], rows_v.at[t], ssems[t]).wait()
        pltpu.async_copy(z.at[src_v.at[pl.ds(j * KC, KC)]], rows_v.at[t],
                         gsem).wait()
        pltpu.sync_copy(rows_v.at[t], acc_sh.at[didx_v.at[pl.ds(j * KC, KC)]],
                        add=True)
    pltpu.make_async_copy(z.at[pl.ds(0, KC)], rows_v.at[2], ssem2).wait()
    plsc.subcore_barrier()
    pltpu.sync_copy(acc_sh.at[pl.ds(sid * RPT, RPT)],
                    out.at[cid, pl.ds(sid * RPT, RPT)])

    @pl.when(sid == NS - 1)
    def _():
        pltpu.sync_copy(acc_sh.at[pl.ds(RPT * NS, RLAST - RPT)],
                        out.at[cid, pl.ds(RPT * NS, RLAST - RPT)])


def _mm_body(x_ref, wt_ref, b_ref, deg_ref, z_ref):
    y = jnp.dot(x_ref[...], wt_ref[...], preferred_element_type=jnp.float32)
    y = y + b_ref[...]
    deg = deg_ref[...]
    d = deg[0] + deg[1] + 1.0
    z_ref[...] = y * lax.rsqrt(d)


def _final_body(acc_ref, z_ref, deg_ref, o_ref):
    deg = deg_ref[...]
    d = deg[0] + deg[1] + 1.0
    norm = lax.rsqrt(d)
    acc = acc_ref[...]
    s = (acc[0] + acc[1] + z_ref[...]) * norm
    o_ref[...] = jnp.maximum(s, 0.0)


def kernel(X, edge_index, W, b):
    src = edge_index[0].astype(jnp.int32)
    dst = edge_index[1].astype(jnp.int32)
    dst3d = dst.reshape(NW, NCHD, KD)

    srcp = src.reshape(NW, EPW)
    dstp2 = dst.reshape(NW, EPW)

    zerosd = jnp.zeros((RLAST, D), jnp.float32)

    degflat = _deg_kernel(dst3d)
    deg = degflat.reshape(NC, N, 1)

    z = pl.pallas_call(
        _mm_body,
        out_shape=jax.ShapeDtypeStruct((N, D), jnp.float32),
    )(X, W.T, b.reshape(1, D), deg)

    acc = _gather_scatter_kernel(z, srcp, dstp2, zerosd)

    out = pl.pallas_call(
        _final_body,
        out_shape=jax.ShapeDtypeStruct((N, D), jnp.float32),
    )(acc, z, deg)
    return out


# speedup vs baseline: 1.1052x; 1.0281x over previous
import functools

import jax
import jax.numpy as jnp
from jax import lax
from jax.experimental import pallas as pl
from jax.experimental.pallas import tpu as pltpu
from jax.experimental.pallas import tpu_sc as plsc

N = 10000
E = 320000
D = 128

NC = 2
NS = 16
NW = NC * NS
EPW = E // NW

KD = 125
NCHD = EPW // KD

KC = 80
EPWP = EPW
NCH = EPWP // KC
NGRP = NCH // 2

RPT = 624
RLAST = N - RPT * (NS - 1)

_mesh = plsc.VectorSubcoreMesh(core_axis_name="c", subcore_axis_name="s")


@functools.partial(
    pl.kernel,
    out_type=jax.ShapeDtypeStruct((NC * N,), jnp.float32),
    mesh=_mesh,
    scratch_types=[
        pltpu.VMEM((NCHD, KD), jnp.int32),
        pltpu.VMEM((128,), jnp.float32),
        pltpu.VMEM((RLAST,), jnp.float32),
        pltpu.VMEM_SHARED((N,), jnp.float32),
        pltpu.SemaphoreType.DMA,
        pltpu.SemaphoreType.DMA,
    ],
)
def _deg_kernel(dst3d, out, idx_v, ones_v, stage_v, deg_sh, dsem0, dsem1):
    cid = lax.axis_index("c")
    sid = lax.axis_index("s")
    wid = cid * NS + sid
    for i in range(128 // 16):
        ones_v[pl.ds(i * 16, 16)] = jnp.ones((16,), jnp.float32)
    for i in range(RLAST // 16):
        stage_v[pl.ds(i * 16, 16)] = jnp.zeros((16,), jnp.float32)
    pltpu.sync_copy(stage_v.at[pl.ds(0, RPT)], deg_sh.at[pl.ds(sid * RPT, RPT)])

    @pl.when(sid == NS - 1)
    def _():
        pltpu.sync_copy(stage_v.at[pl.ds(RPT, RLAST - RPT)],
                        deg_sh.at[pl.ds(RPT * NS, RLAST - RPT)])

    pltpu.sync_copy(dst3d.at[wid], idx_v)
    plsc.subcore_barrier()
    dsems = (dsem0, dsem1)

    def step(jj, carry):
        for h in range(2):
            j = jj * 2 + h

            @pl.when(j >= 2)
            def _():
                pltpu.make_async_copy(out.at[pl.ds(0, KD)],
                                      ones_v.at[pl.ds(0, KD)], dsems[h]).wait()
            pltpu.async_copy(ones_v.at[pl.ds(0, KD)],
                             deg_sh.at[idx_v.at[j]], dsems[h], add=True)
        return carry

    lax.fori_loop(0, NCHD // 2, step, 0)
    pltpu.make_async_copy(out.at[pl.ds(0, KD)], ones_v.at[pl.ds(0, KD)],
                          dsem0).wait()
    pltpu.make_async_copy(out.at[pl.ds(0, KD)], ones_v.at[pl.ds(0, KD)],
                          dsem1).wait()
    plsc.subcore_barrier()
    pltpu.sync_copy(deg_sh.at[pl.ds(sid * RPT, RPT)], stage_v.at[pl.ds(0, RPT)])
    pltpu.sync_copy(stage_v.at[pl.ds(0, RPT)],
                    out.at[pl.ds(cid * N + sid * RPT, RPT)])

    @pl.when(sid == NS - 1)
    def _():
        pltpu.sync_copy(deg_sh.at[pl.ds(RPT * NS, RLAST - RPT)],
                        stage_v.at[pl.ds(RPT, RLAST - RPT)])
        pltpu.sync_copy(stage_v.at[pl.ds(RPT, RLAST - RPT)],
                        out.at[pl.ds(cid * N + RPT * NS, RLAST - RPT)])


@functools.partial(
    pl.kernel,
    out_type=jax.ShapeDtypeStruct((NC, N, D), jnp.float32),
    mesh=_mesh,
    scratch_types=[
        pltpu.VMEM((EPWP,), jnp.int32),
        pltpu.VMEM((EPWP,), jnp.int32),
        pltpu.VMEM((3, KC, D), jnp.float32),
        pltpu.VMEM_SHARED((N, D), jnp.float32),
        pltpu.SemaphoreType.DMA,
        pltpu.SemaphoreType.DMA,
        pltpu.SemaphoreType.DMA,
        pltpu.SemaphoreType.DMA,
    ],
)
def _gather_scatter_kernel(z, srcp, dstp2, zerosd, out,
                           src_v, didx_v, rows_v, acc_sh,
                           gsem, ssem0, ssem1, ssem2):
    cid = lax.axis_index("c")
    sid = lax.axis_index("s")
    wid = cid * NS + sid
    pltpu.sync_copy(zerosd.at[pl.ds(0, RPT)], acc_sh.at[pl.ds(sid * RPT, RPT)])

    @pl.when(sid == NS - 1)
    def _():
        pltpu.sync_copy(zerosd.at[pl.ds(RPT, RLAST - RPT)],
                        acc_sh.at[pl.ds(RPT * NS, RLAST - RPT)])

    pltpu.sync_copy(srcp.at[wid], src_v)
    pltpu.sync_copy(dstp2.at[wid], didx_v)
    plsc.subcore_barrier()

    ssems = (ssem0, ssem1, ssem2)
    NSLOT = 3

    def body(gg, carry):
        gds = []
        for h in range(NSLOT):
            g = gg * NSLOT + h
            @pl.when(g >= NSLOT)
            def _():
                pltpu.make_async_copy(z.at[pl.ds(0, KC)], rows_v.at[h],
                                      ssems[h]).wait()
            off = pl.multiple_of(g * KC, 8)
            gds.append(pltpu.async_copy(
                z.at[src_v.at[pl.ds(off, KC)]], rows_v.at[h], gsem))
        for h in range(NSLOT):
            g = gg * NSLOT + h
            goff = pl.multiple_of(g * KC, 8)
            gds[h].wait()
            pltpu.async_copy(rows_v.at[h],
                             acc_sh.at[didx_v.at[pl.ds(goff, KC)]],
                             ssems[h], add=True)
        return carry

    nbody = NCH // NSLOT
    lax.fori_loop(0, nbody, body, 0)
    for t in range(NCH - nbody * NSLOT):
        j = nbody * NSLOT + t
        pltpu.make_async_copy(z.at[pl.ds(0, KC)], rows_v.at[t], ssems[t]).wait()
        pltpu.async_copy(z.at[src_v.at[pl.ds(j * KC, KC)]], rows_v.at[t],
                         gsem).wait()
        pltpu.sync_copy(rows_v.at[t], acc_sh.at[didx_v.at[pl.ds(j * KC, KC)]],
                        add=True)
    pltpu.make_async_copy(z.at[pl.ds(0, KC)], rows_v.at[2], ssem2).wait()
    plsc.subcore_barrier()
    pltpu.sync_copy(acc_sh.at[pl.ds(sid * RPT, RPT)],
                    out.at[cid, pl.ds(sid * RPT, RPT)])

    @pl.when(sid == NS - 1)
    def _():
        pltpu.sync_copy(acc_sh.at[pl.ds(RPT * NS, RLAST - RPT)],
                        out.at[cid, pl.ds(RPT * NS, RLAST - RPT)])


def _mm_body(x_ref, wt_ref, b_ref, deg_ref, z_ref):
    y = jnp.dot(x_ref[...], wt_ref[...], preferred_element_type=jnp.float32)
    y = y + b_ref[...]
    deg = deg_ref[...]
    d = deg[0] + deg[1] + 1.0
    z_ref[...] = y * lax.rsqrt(d)


def _final_body(acc_ref, z_ref, deg_ref, o_ref):
    deg = deg_ref[...]
    d = deg[0] + deg[1] + 1.0
    norm = lax.rsqrt(d)
    acc = acc_ref[...]
    s = (acc[0] + acc[1] + z_ref[...]) * norm
    o_ref[...] = jnp.maximum(s, 0.0)


def kernel(X, edge_index, W, b):
    src = edge_index[0].astype(jnp.int32)
    dst = edge_index[1].astype(jnp.int32)
    dst3d = dst.reshape(NW, NCHD, KD)

    srcp = src.reshape(NW, EPW)
    dstp2 = dst.reshape(NW, EPW)

    zerosd = jnp.zeros((RLAST, D), jnp.float32)

    degflat = _deg_kernel(dst3d)
    deg = degflat.reshape(NC, N, 1)

    z = pl.pallas_call(
        _mm_body,
        out_shape=jax.ShapeDtypeStruct((N, D), jnp.float32),
    )(X, W.T, b.reshape(1, D), deg)

    acc = _gather_scatter_kernel(z, srcp, dstp2, zerosd)

    out = pl.pallas_call(
        _final_body,
        out_shape=jax.ShapeDtypeStruct((N, D), jnp.float32),
    )(acc, z, deg)
    return out
